# Initial kernel scaffold; baseline (speedup 1.0000x reference)
import jax
import jax.numpy as jnp
from jax.experimental import pallas as pl


def kernel(items, A0, A1, A2, A3, A4, B0, B1, B2, B3, B4, t):
    raise NotImplementedError("write your pallas kernel here")


# speedup vs baseline: 50.1621x; 50.1621x over previous
import functools

import jax
import jax.numpy as jnp
from jax import lax
from jax.experimental import pallas as pl
from jax.experimental.pallas import tpu as pltpu
from jax.experimental.pallas import tpu_sc as plsc

_PRIMES = (4993, 4999, 5003, 5009, 5011)
_ALPHA = 0.0001
_N = 16384
_LANES = 16
_NC, _NS = 2, 16
_NW = _NC * _NS
_CHUNK = _N // _NW
_LN2 = 0.6931471805599453


def _neg_log(x):
    bits = plsc.bitcast(x, jnp.int32)
    e = ((bits >> 23) & 0xFF) - 127
    m = plsc.bitcast((bits & 0x7FFFFF) | 0x3F800000, jnp.float32)
    hi = m >= 1.5
    m = jnp.where(hi, 0.5 * m, m)
    e = jnp.where(hi, e + 1, e)
    s = (m - 1.0) / (m + 1.0)
    s2 = s * s
    poly = 1.0 + s2 * ((1.0 / 3.0) + s2 * ((1.0 / 5.0) + s2 * (1.0 / 7.0)))
    lnm = (2.0 * s) * poly
    return -(e.astype(jnp.float32) * _LN2 + lnm)


def _mod(it, it_f, p):
    q = (it_f * (1.0 / p)).astype(jnp.int32)
    r = it - q * p
    r = jnp.where(r < 0, r + p, r)
    r = jnp.where(r >= p, r - p, r)
    return r


def _build():
    scratch = [pltpu.VMEM((_CHUNK,), jnp.int32)]
    scratch += [pltpu.VMEM((p,), jnp.float32) for p in _PRIMES]
    scratch += [pltpu.VMEM((p,), jnp.float32) for p in _PRIMES]
    scratch += [pltpu.VMEM((_LANES,), jnp.float32)]
    scratch += [pltpu.VMEM((_CHUNK,), jnp.float32)]
    scratch += [pltpu.SemaphoreType.DMA]
    mesh = plsc.VectorSubcoreMesh(core_axis_name="c", subcore_axis_name="s")

    @functools.partial(
        pl.kernel,
        out_type=jax.ShapeDtypeStruct((_N,), jnp.float32),
        mesh=mesh,
        scratch_types=scratch,
        compiler_params=pltpu.CompilerParams(needs_layout_passes=False),
    )
    def sc_kernel(items_h, a0h, a1h, a2h, a3h, a4h, b0h, b1h, b2h, b3h, b4h,
                  c_h, out_h,
                  it_v, av0, av1, av2, av3, av4, bv0, bv1, bv2, bv3, bv4,
                  c_v, out_v, sem):
        wid = lax.axis_index("s") * _NC + lax.axis_index("c")
        base = wid * _CHUNK
        avs = (av0, av1, av2, av3, av4)
        bvs = (bv0, bv1, bv2, bv3, bv4)
        copies = [pltpu.async_copy(items_h.at[pl.ds(base, _CHUNK)], it_v, sem),
                  pltpu.async_copy(c_h, c_v, sem)]
        for src, dst in zip((a0h, a1h, a2h, a3h, a4h), avs):
            copies.append(pltpu.async_copy(src, dst, sem))
        for src, dst in zip((b0h, b1h, b2h, b3h, b4h), bvs):
            copies.append(pltpu.async_copy(src, dst, sem))
        for cp in copies:
            cp.wait()
        cvec = c_v[...]

        def body(i, carry):
            off = i * _LANES
            it = it_v[pl.ds(off, _LANES)]
            it_f = it.astype(jnp.float32)
            best = None
            for p, av, bv in zip(_PRIMES, avs, bvs):
                r = _mod(it, it_f, p)
                a = plsc.load_gather(av, [r])
                b = plsc.load_gather(bv, [r])
                d = (1.0 - _ALPHA) * b - _ALPHA * a
                best = d if best is None else jnp.maximum(best, d)
            out_v[pl.ds(off, _LANES)] = _neg_log(best + cvec)
            return carry

        lax.fori_loop(0, _CHUNK // _LANES, body, jnp.int32(0))
        pltpu.async_copy(out_v, out_h.at[pl.ds(base, _CHUNK)], sem).wait()

    return sc_kernel


_SC_KERNEL = _build()


def kernel(items, A0, A1, A2, A3, A4, B0, B1, B2, B3, B4, t):
    c16 = jnp.broadcast_to((t + 1.0) * _ALPHA, (_LANES,)).astype(jnp.float32)
    return _SC_KERNEL(items, A0, A1, A2, A3, A4, B0, B1, B2, B3, B4, c16)
